# Initial kernel scaffold; baseline (speedup 1.0000x reference)
import jax
import jax.numpy as jnp
from jax.experimental import pallas as pl


def kernel(input, table):
    raise NotImplementedError("write your pallas kernel here")


# speedup vs baseline: 1.0223x; 1.0223x over previous
import functools

import jax
import jax.numpy as jnp
from jax import lax
from jax.experimental import pallas as pl
from jax.experimental.pallas import tpu as pltpu
from jax.experimental.pallas import tpu_sc as plsc

VOCAB = 1000000
EMBED_DIM = 32
BATCH = 16384
HIST = 50

TOTAL = BATCH * HIST
STEP = 128
N_ROWS = TOTAL // STEP


def _make_kernel(n_workers: int, nc: int):
    nstep = N_ROWS // n_workers
    mesh = plsc.VectorSubcoreMesh(core_axis_name="c", subcore_axis_name="s")

    @functools.partial(
        pl.kernel,
        out_type=jax.ShapeDtypeStruct((TOTAL, EMBED_DIM), jnp.float32),
        mesh=mesh,
        scratch_types=[
            pltpu.VMEM((nstep, STEP), jnp.int32),
            pltpu.VMEM((STEP, EMBED_DIM), jnp.float32),
            pltpu.SemaphoreType.DMA,
        ],
        compiler_params=pltpu.CompilerParams(use_tc_tiling_on_sc=False),
    )
    def k(idx_hbm, table_hbm, out_hbm, idx_v, rows_v, gsem):
        wid = lax.axis_index("s") * nc + lax.axis_index("c")
        pltpu.sync_copy(idx_hbm.at[pl.ds(wid * nstep, nstep)], idx_v)
        base = wid * nstep * STEP

        def step(j, carry):
            pltpu.async_copy(table_hbm.at[idx_v.at[j]], rows_v, gsem).wait()
            pltpu.sync_copy(rows_v, out_hbm.at[pl.ds(base + j * STEP, STEP)])
            return carry

        lax.fori_loop(0, nstep, step, 0)

    return k


def kernel(input, table):
    info = plsc.get_sparse_core_info()
    n_workers = info.num_cores * info.num_subcores
    idx = input.reshape(N_ROWS, STEP).astype(jnp.int32)
    out = _make_kernel(n_workers, info.num_cores)(idx, table)
    return out.reshape(BATCH, HIST, EMBED_DIM)


# speedup vs baseline: 1.1119x; 1.0876x over previous
import functools

import jax
import jax.numpy as jnp
from jax import lax
from jax.experimental import pallas as pl
from jax.experimental.pallas import tpu as pltpu
from jax.experimental.pallas import tpu_sc as plsc

VOCAB = 1000000
EMBED_DIM = 32
BATCH = 16384
HIST = 50

TOTAL = BATCH * HIST
STEP = 128
N_ROWS = TOTAL // STEP


NBUF = 8
LOOKAHEAD = 6


def _make_kernel(n_workers: int, nc: int):
    nstep = N_ROWS // n_workers
    mesh = plsc.VectorSubcoreMesh(core_axis_name="c", subcore_axis_name="s")

    @functools.partial(
        pl.kernel,
        out_type=jax.ShapeDtypeStruct((TOTAL, EMBED_DIM), jnp.float32),
        mesh=mesh,
        scratch_types=[
            pltpu.VMEM((nstep, STEP), jnp.int32),
            pltpu.VMEM((NBUF, STEP, EMBED_DIM), jnp.float32),
            pltpu.SemaphoreType.DMA((NBUF,)),
            pltpu.SemaphoreType.DMA((NBUF,)),
        ],
        compiler_params=pltpu.CompilerParams(use_tc_tiling_on_sc=False),
    )
    def k(idx_hbm, table_hbm, out_hbm, idx_v, rows_v, gsem, wsem):
        wid = lax.axis_index("s") * nc + lax.axis_index("c")
        pltpu.sync_copy(idx_hbm.at[pl.ds(wid * nstep, nstep)], idx_v)
        base = wid * nstep * STEP

        for b in range(LOOKAHEAD):
            pltpu.async_copy(table_hbm.at[idx_v.at[b]], rows_v.at[b],
                             gsem.at[b])

        def block(j0, carry):
            for b in range(NBUF):
                j = j0 + b
                jn = j + LOOKAHEAD
                bn = (b + LOOKAHEAD) % NBUF

                @pl.when(jn < nstep)
                def _():
                    @pl.when(jn >= NBUF)
                    def _():
                        pltpu.make_async_copy(
                            rows_v.at[bn],
                            out_hbm.at[pl.ds(base, STEP)],
                            wsem.at[bn]).wait()
                    pltpu.async_copy(table_hbm.at[idx_v.at[jn]],
                                     rows_v.at[bn], gsem.at[bn])

                pltpu.make_async_copy(
                    table_hbm.at[idx_v.at[j]], rows_v.at[b],
                    gsem.at[b]).wait()
                pltpu.async_copy(rows_v.at[b],
                                 out_hbm.at[pl.ds(base + j * STEP, STEP)],
                                 wsem.at[b])
            return carry

        lax.fori_loop(0, nstep // NBUF, lambda i, c: block(i * NBUF, c), 0)

        for b in range(NBUF):
            pltpu.make_async_copy(rows_v.at[b],
                                  out_hbm.at[pl.ds(base, STEP)],
                                  wsem.at[b]).wait()

    return k


def kernel(input, table):
    info = plsc.get_sparse_core_info()
    n_workers = info.num_cores * info.num_subcores
    idx = input.reshape(N_ROWS, STEP).astype(jnp.int32)
    out = _make_kernel(n_workers, info.num_cores)(idx, table)
    return out.reshape(BATCH, HIST, EMBED_DIM)


# speedup vs baseline: 1.7876x; 1.6077x over previous
import functools

import jax
import jax.numpy as jnp
from jax import lax
from jax.experimental import pallas as pl
from jax.experimental.pallas import tpu as pltpu
from jax.experimental.pallas import tpu_sc as plsc

VOCAB = 1000000
EMBED_DIM = 32
BATCH = 16384
HIST = 50

NBUF = 8
LOOKAHEAD = 6


def _make_kernel(n_workers: int, nc: int):
    nstep = BATCH // n_workers
    mesh = plsc.VectorSubcoreMesh(core_axis_name="c", subcore_axis_name="s")

    @functools.partial(
        pl.kernel,
        out_type=jax.ShapeDtypeStruct((BATCH, HIST, EMBED_DIM), jnp.float32),
        mesh=mesh,
        scratch_types=[
            pltpu.VMEM((nstep, HIST), jnp.int32),
            pltpu.VMEM((NBUF, HIST, EMBED_DIM), jnp.float32),
            pltpu.SemaphoreType.DMA((NBUF,)),
            pltpu.SemaphoreType.DMA((NBUF,)),
        ],
        compiler_params=pltpu.CompilerParams(use_tc_tiling_on_sc=False),
    )
    def k(idx_hbm, table_hbm, out_hbm, idx_v, rows_v, gsem, wsem):
        wid = lax.axis_index("s") * nc + lax.axis_index("c")
        base = wid * nstep
        pltpu.sync_copy(idx_hbm.at[pl.ds(base, nstep)], idx_v)

        for b in range(LOOKAHEAD):
            pltpu.async_copy(table_hbm.at[idx_v.at[b]], rows_v.at[b],
                             gsem.at[b])

        def block(j0, carry):
            for b in range(NBUF):
                j = j0 + b
                jn = j + LOOKAHEAD
                bn = (b + LOOKAHEAD) % NBUF

                @pl.when(jn < nstep)
                def _():
                    @pl.when(jn >= NBUF)
                    def _():
                        pltpu.make_async_copy(
                            rows_v.at[bn], out_hbm.at[base],
                            wsem.at[bn]).wait()
                    pltpu.async_copy(table_hbm.at[idx_v.at[jn]],
                                     rows_v.at[bn], gsem.at[bn])

                pltpu.make_async_copy(
                    table_hbm.at[idx_v.at[j]], rows_v.at[b],
                    gsem.at[b]).wait()
                pltpu.async_copy(rows_v.at[b], out_hbm.at[base + j],
                                 wsem.at[b])
            return carry

        lax.fori_loop(0, nstep // NBUF, lambda i, c: block(i * NBUF, c), 0)

        for b in range(NBUF):
            pltpu.make_async_copy(rows_v.at[b], out_hbm.at[base],
                                  wsem.at[b]).wait()

    return k


def kernel(input, table):
    info = plsc.get_sparse_core_info()
    n_workers = info.num_cores * info.num_subcores
    return _make_kernel(n_workers, info.num_cores)(input.astype(jnp.int32),
                                                   table)
